# baseline (speedup 1.0000x reference)
import functools
import math

import jax
import jax.numpy as jnp
from jax import lax
from jax.experimental import pallas as pl
from jax.experimental.pallas import tpu as pltpu
from jax.experimental.pallas import tpu_sc as plsc

B = 4
SEQ = 2048
D = 768
SCALE = float(math.sqrt(D))

NC = 2
NS = 16
NW = NC * NS
S_PER_W = SEQ // NW
CH = 32
NCHUNK = B * (S_PER_W // CH)
NBUF = 3
LANES = 16
DJ = D // LANES


def _body(tok_hbm, tokens_hbm, pos_hbm, out_hbm,
          idx_v, pos_v, rows_v, g_sem, o_sem, p_sem):
    wid = lax.axis_index("s") * NC + lax.axis_index("c")
    s0 = wid * S_PER_W

    pos_cp = pltpu.make_async_copy(pos_hbm.at[pl.ds(s0, S_PER_W)], pos_v, p_sem)
    pos_cp.start()

    for g in range(NCHUNK):
        b, h = divmod(g, S_PER_W // CH)
        pltpu.sync_copy(tokens_hbm.at[b, pl.ds(s0 + h * CH, CH)], idx_v.at[g])

    def g_desc(g):
        buf = g % NBUF
        return pltpu.make_async_copy(
            tok_hbm.at[idx_v.at[g]], rows_v.at[buf], g_sem.at[buf])

    def o_desc(g):
        buf = g % NBUF
        b, h = divmod(g, S_PER_W // CH)
        return pltpu.make_async_copy(
            rows_v.at[buf], out_hbm.at[b, pl.ds(s0 + h * CH, CH)], o_sem.at[buf])

    g_desc(0).start()
    g_desc(1).start()
    pos_cp.wait()

    for g in range(NCHUNK):
        buf = g % NBUF
        h = g % (S_PER_W // CH)
        g_desc(g).wait()

        def row_body(r, carry, buf=buf, h=h):
            for j in range(DJ):
                sl = pl.ds(j * LANES, LANES)
                rows_v[buf, r, sl] = rows_v[buf, r, sl] * SCALE + pos_v[h * CH + r, sl]
            return carry

        lax.fori_loop(0, CH, row_body, 0)

        o_desc(g).start()
        if g + 2 < NCHUNK:
            if g >= 1:
                o_desc(g - 1).wait()
            g_desc(g + 2).start()

    o_desc(NCHUNK - 2).wait()
    o_desc(NCHUNK - 1).wait()


@functools.partial(
    pl.kernel,
    out_type=jax.ShapeDtypeStruct((B, SEQ, D), jnp.float32),
    mesh=plsc.VectorSubcoreMesh(core_axis_name="c", subcore_axis_name="s"),
    scratch_types=[
        pltpu.VMEM((NCHUNK, CH), jnp.int32),
        pltpu.VMEM((S_PER_W, D), jnp.float32),
        pltpu.VMEM((NBUF, CH, D), jnp.float32),
        pltpu.SemaphoreType.DMA((NBUF,)),
        pltpu.SemaphoreType.DMA((NBUF,)),
        pltpu.SemaphoreType.DMA,
    ],
)
def _embed_kernel(tok_hbm, tokens_hbm, pos_hbm, out_hbm,
                  idx_v, pos_v, rows_v, g_sem, o_sem, p_sem):
    _body(tok_hbm, tokens_hbm, pos_hbm, out_hbm,
          idx_v, pos_v, rows_v, g_sem, o_sem, p_sem)


def kernel(tokens, tok_table, pos_table):
    return _embed_kernel(tok_table, tokens.astype(jnp.int32), pos_table)


# speedup vs baseline: 1.3251x; 1.3251x over previous
import functools
import math

import jax
import jax.numpy as jnp
from jax import lax
from jax.experimental import pallas as pl
from jax.experimental.pallas import tpu as pltpu
from jax.experimental.pallas import tpu_sc as plsc

B = 4
SEQ = 2048
D = 768
SCALE = float(math.sqrt(D))

NC = 2
NS = 16
NW = NC * NS
S_PER_W = SEQ // NW
CH = 32
NCHUNK = B * (S_PER_W // CH)
NBUF = 3
LANES = 16
DJ = D // LANES


def _body(tok_hbm, tokens_hbm, pos_hbm, out_hbm,
          idx_v, pos_v, rows_v, g_sem, o_sem, p_sem):
    wid = lax.axis_index("s") * NC + lax.axis_index("c")
    s0 = wid * S_PER_W

    pos_cp = pltpu.make_async_copy(pos_hbm.at[pl.ds(s0, S_PER_W)], pos_v, p_sem)
    pos_cp.start()

    for g in range(NCHUNK):
        b, h = divmod(g, S_PER_W // CH)
        pltpu.sync_copy(tokens_hbm.at[b, pl.ds(s0 + h * CH, CH)], idx_v.at[g])

    def g_desc(g):
        buf = g % NBUF
        return pltpu.make_async_copy(
            tok_hbm.at[idx_v.at[g]], rows_v.at[buf], g_sem.at[buf])

    def o_desc(g):
        buf = g % NBUF
        b, h = divmod(g, S_PER_W // CH)
        return pltpu.make_async_copy(
            rows_v.at[buf], out_hbm.at[b, pl.ds(s0 + h * CH, CH)], o_sem.at[buf])

    g_desc(0).start()
    g_desc(1).start()
    pos_cp.wait()

    for g in range(NCHUNK):
        buf = g % NBUF
        h = g % (S_PER_W // CH)
        g_desc(g).wait()

        @plsc.parallel_loop(0, CH, 1, unroll=2)
        def _row_body(r, buf=buf, h=h):
            for j in range(DJ):
                sl = pl.ds(j * LANES, LANES)
                rows_v[buf, r, sl] = rows_v[buf, r, sl] * SCALE + pos_v[h * CH + r, sl]

        o_desc(g).start()
        if g + 2 < NCHUNK:
            if g >= 1:
                o_desc(g - 1).wait()
            g_desc(g + 2).start()

    o_desc(NCHUNK - 2).wait()
    o_desc(NCHUNK - 1).wait()


@functools.partial(
    pl.kernel,
    out_type=jax.ShapeDtypeStruct((B, SEQ, D), jnp.float32),
    mesh=plsc.VectorSubcoreMesh(core_axis_name="c", subcore_axis_name="s"),
    scratch_types=[
        pltpu.VMEM((NCHUNK, CH), jnp.int32),
        pltpu.VMEM((S_PER_W, D), jnp.float32),
        pltpu.VMEM((NBUF, CH, D), jnp.float32),
        pltpu.SemaphoreType.DMA((NBUF,)),
        pltpu.SemaphoreType.DMA((NBUF,)),
        pltpu.SemaphoreType.DMA,
    ],
)
def _embed_kernel(tok_hbm, tokens_hbm, pos_hbm, out_hbm,
                  idx_v, pos_v, rows_v, g_sem, o_sem, p_sem):
    _body(tok_hbm, tokens_hbm, pos_hbm, out_hbm,
          idx_v, pos_v, rows_v, g_sem, o_sem, p_sem)


def kernel(tokens, tok_table, pos_table):
    return _embed_kernel(tok_table, tokens.astype(jnp.int32), pos_table)


# speedup vs baseline: 1.3980x; 1.0550x over previous
import functools
import math

import jax
import jax.numpy as jnp
from jax import lax
from jax.experimental import pallas as pl
from jax.experimental.pallas import tpu as pltpu
from jax.experimental.pallas import tpu_sc as plsc

B = 4
SEQ = 2048
D = 768
SCALE = float(math.sqrt(D))

NC = 2
NS = 16
NW = NC * NS
S_PER_W = SEQ // NW
CH = 16
NH = S_PER_W // CH
NCHUNK = B * NH
NBUF = 6
AHEAD = 4
LANES = 16
DJ = D // LANES


def _body(tok_hbm, tokens_hbm, pos_hbm, out_hbm,
          idx_refs, pos_v, rows_v, g_sem, o_sem, p_sem, i_sem):
    wid = lax.axis_index("s") * NC + lax.axis_index("c")
    s0 = wid * S_PER_W

    def pos_cp(h):
        return pltpu.make_async_copy(
            pos_hbm.at[pl.ds(s0 + h * CH, CH)], pos_v.at[h % 2], p_sem.at[h % 2])

    pos_cp(0).start()

    idx_cps = []
    for g in range(NCHUNK):
        h, b = divmod(g, B)
        cp = pltpu.make_async_copy(
            tokens_hbm.at[b, pl.ds(s0 + h * CH, CH)], idx_refs[g], i_sem)
        cp.start()
        idx_cps.append(cp)
    for cp in idx_cps:
        cp.wait()

    def g_desc(g):
        buf = g % NBUF
        return pltpu.make_async_copy(
            tok_hbm.at[idx_refs[g]], rows_v.at[buf], g_sem.at[buf])

    def o_desc(g):
        buf = g % NBUF
        h, b = divmod(g, B)
        return pltpu.make_async_copy(
            rows_v.at[buf], out_hbm.at[b, pl.ds(s0 + h * CH, CH)], o_sem.at[buf])

    for g in range(AHEAD):
        g_desc(g).start()

    for g in range(NCHUNK):
        buf = g % NBUF
        h = g // B
        if g % B == 0:
            if h + 1 < NH:
                pos_cp(h + 1).start()
            pos_cp(h).wait()
        g_desc(g).wait()

        @plsc.parallel_loop(0, CH, 1, unroll=1)
        def _row_body(r, buf=buf, hp=h % 2):
            for j in range(DJ):
                sl = pl.ds(j * LANES, LANES)
                rows_v[buf, r, sl] = rows_v[buf, r, sl] * SCALE + pos_v[hp, r, sl]

        o_desc(g).start()
        if g + AHEAD < NCHUNK:
            if g - (NBUF - AHEAD) >= 0:
                o_desc(g - (NBUF - AHEAD)).wait()
            g_desc(g + AHEAD).start()

    for g in range(NCHUNK - AHEAD - (NBUF - AHEAD), NCHUNK):
        if g >= 0:
            o_desc(g).wait()


@functools.partial(
    pl.kernel,
    out_type=jax.ShapeDtypeStruct((B, SEQ, D), jnp.float32),
    mesh=plsc.VectorSubcoreMesh(core_axis_name="c", subcore_axis_name="s"),
    scratch_types=[
        [pltpu.VMEM((CH,), jnp.int32) for _ in range(NCHUNK)],
        pltpu.VMEM((2, CH, D), jnp.float32),
        pltpu.VMEM((NBUF, CH, D), jnp.float32),
        pltpu.SemaphoreType.DMA((NBUF,)),
        pltpu.SemaphoreType.DMA((NBUF,)),
        pltpu.SemaphoreType.DMA((2,)),
        pltpu.SemaphoreType.DMA,
    ],
)
def _embed_kernel(tok_hbm, tokens_hbm, pos_hbm, out_hbm,
                  idx_refs, pos_v, rows_v, g_sem, o_sem, p_sem, i_sem):
    _body(tok_hbm, tokens_hbm, pos_hbm, out_hbm,
          idx_refs, pos_v, rows_v, g_sem, o_sem, p_sem, i_sem)


def kernel(tokens, tok_table, pos_table):
    return _embed_kernel(tok_table, tokens.astype(jnp.int32), pos_table)


# speedup vs baseline: 1.5791x; 1.1296x over previous
import functools
import math

import jax
import jax.numpy as jnp
from jax import lax
from jax.experimental import pallas as pl
from jax.experimental.pallas import tpu as pltpu
from jax.experimental.pallas import tpu_sc as plsc

B = 4
SEQ = 2048
D = 768
SCALE = float(math.sqrt(D))

NC = 2
NS = 16
NW = NC * NS
S_PER_W = SEQ // NW
CH = 8
NH = S_PER_W // CH
NCHUNK = B * NH
NGRP = 3
NBUF = NGRP * B
LANES = 16
DJ = D // LANES


def _body(tok_hbm, tokens_hbm, pos_hbm, out_hbm,
          idx_refs, pos_v, rows_v, g_sem, o_sem, p_sem, i_sem):
    wid = lax.axis_index("s") * NC + lax.axis_index("c")
    s0 = wid * S_PER_W

    def pos_cp(h):
        return pltpu.make_async_copy(
            pos_hbm.at[pl.ds(s0 + h * CH, CH)], pos_v.at[h % 2], p_sem.at[h % 2])

    pos_cp(0).start()

    idx_cps = []
    for g in range(NCHUNK):
        h, b = divmod(g, B)
        cp = pltpu.make_async_copy(
            tokens_hbm.at[b, pl.ds(s0 + h * CH, CH)], idx_refs[g], i_sem)
        cp.start()
        idx_cps.append(cp)
    for cp in idx_cps:
        cp.wait()

    def buf(g):
        h, b = divmod(g, B)
        return (h % NGRP) * B + b

    def g_desc(g):
        return pltpu.make_async_copy(
            tok_hbm.at[idx_refs[g]], rows_v.at[buf(g)], g_sem.at[buf(g)])

    def o_desc(g):
        h, b = divmod(g, B)
        return pltpu.make_async_copy(
            rows_v.at[buf(g)], out_hbm.at[b, pl.ds(s0 + h * CH, CH)],
            o_sem.at[buf(g)])

    for g in range(2 * B):
        g_desc(g).start()

    for h in range(NH):
        if h >= 2:
            for b in range(B):
                o_desc((h - 2) * B + b).wait()
        if 2 <= h + 1 < NH:
            for b in range(B):
                g_desc((h + 1) * B + b).start()
        if h + 1 < NH:
            pos_cp(h + 1).start()
        pos_cp(h).wait()
        for b in range(B):
            g_desc(h * B + b).wait()

        base = (h % NGRP) * B

        @plsc.parallel_loop(0, DJ, 1, unroll=1)
        def _col_body(j, base=base, hp=h % 2):
            sl = pl.ds(j * LANES, LANES)
            for r in range(CH):
                p = pos_v[hp, r, sl]
                for b in range(B):
                    rows_v[base + b, r, sl] = rows_v[base + b, r, sl] * SCALE + p

        for b in range(B):
            o_desc(h * B + b).start()

    for h in range(NH - 2, NH):
        for b in range(B):
            o_desc(h * B + b).wait()


@functools.partial(
    pl.kernel,
    out_type=jax.ShapeDtypeStruct((B, SEQ, D), jnp.float32),
    mesh=plsc.VectorSubcoreMesh(core_axis_name="c", subcore_axis_name="s"),
    scratch_types=[
        [pltpu.VMEM((CH,), jnp.int32) for _ in range(NCHUNK)],
        pltpu.VMEM((2, CH, D), jnp.float32),
        pltpu.VMEM((NBUF, CH, D), jnp.float32),
        pltpu.SemaphoreType.DMA((NBUF,)),
        pltpu.SemaphoreType.DMA((NBUF,)),
        pltpu.SemaphoreType.DMA((2,)),
        pltpu.SemaphoreType.DMA,
    ],
)
def _embed_kernel(tok_hbm, tokens_hbm, pos_hbm, out_hbm,
                  idx_refs, pos_v, rows_v, g_sem, o_sem, p_sem, i_sem):
    _body(tok_hbm, tokens_hbm, pos_hbm, out_hbm,
          idx_refs, pos_v, rows_v, g_sem, o_sem, p_sem, i_sem)


def kernel(tokens, tok_table, pos_table):
    return _embed_kernel(tok_table, tokens.astype(jnp.int32), pos_table)


# speedup vs baseline: 1.5802x; 1.0007x over previous
import functools
import math

import jax
import jax.numpy as jnp
from jax import lax
from jax.experimental import pallas as pl
from jax.experimental.pallas import tpu as pltpu
from jax.experimental.pallas import tpu_sc as plsc

B = 4
SEQ = 2048
D = 768
SCALE = float(math.sqrt(D))

NC = 2
NS = 16
NW = NC * NS
S_PER_W = SEQ // NW
CH = 8
NH = S_PER_W // CH
NCHUNK = B * NH
NGRP = 3
NBUF = NGRP * B
LANES = 16
DJ = D // LANES


def _body(tok_hbm, tokens_hbm, pos_hbm, out_hbm,
          idx_refs, pos_v, rows_v, g_sem, o_sem, p_sem, i_sem):
    wid = lax.axis_index("s") * NC + lax.axis_index("c")
    s0 = wid * S_PER_W

    def pos_cp(h):
        return pltpu.make_async_copy(
            pos_hbm.at[pl.ds(s0 + h * CH, CH)], pos_v.at[h % 2], p_sem.at[h % 2])

    pos_cp(0).start()

    idx_cps = []
    for g in range(NCHUNK):
        h, b = divmod(g, B)
        cp = pltpu.make_async_copy(
            tokens_hbm.at[b, pl.ds(s0 + h * CH, CH)], idx_refs[g], i_sem)
        cp.start()
        idx_cps.append(cp)
    for cp in idx_cps[:2 * B]:
        cp.wait()

    def buf(g):
        h, b = divmod(g, B)
        return (h % NGRP) * B + b

    def g_desc(g):
        return pltpu.make_async_copy(
            tok_hbm.at[idx_refs[g]], rows_v.at[buf(g)], g_sem.at[buf(g)])

    def o_desc(g):
        h, b = divmod(g, B)
        return pltpu.make_async_copy(
            rows_v.at[buf(g)], out_hbm.at[b, pl.ds(s0 + h * CH, CH)],
            o_sem.at[buf(g)])

    for g in range(2 * B):
        g_desc(g).start()
    for cp in idx_cps[2 * B:]:
        cp.wait()

    for h in range(NH):
        if h >= 2:
            for b in range(B):
                o_desc((h - 2) * B + b).wait()
        if 2 <= h + 1 < NH:
            for b in range(B):
                g_desc((h + 1) * B + b).start()
        if h + 1 < NH:
            pos_cp(h + 1).start()
        pos_cp(h).wait()
        for b in range(B):
            g_desc(h * B + b).wait()

        base = (h % NGRP) * B

        @plsc.parallel_loop(0, DJ, 1, unroll=1)
        def _col_body(j, base=base, hp=h % 2):
            sl = pl.ds(j * LANES, LANES)
            for r in range(CH):
                p = pos_v[hp, r, sl]
                for b in range(B):
                    rows_v[base + b, r, sl] = rows_v[base + b, r, sl] * SCALE + p

        for b in range(B):
            o_desc(h * B + b).start()

    for h in range(NH - 2, NH):
        for b in range(B):
            o_desc(h * B + b).wait()


@functools.partial(
    pl.kernel,
    out_type=jax.ShapeDtypeStruct((B, SEQ, D), jnp.float32),
    mesh=plsc.VectorSubcoreMesh(core_axis_name="c", subcore_axis_name="s"),
    scratch_types=[
        [pltpu.VMEM((CH,), jnp.int32) for _ in range(NCHUNK)],
        pltpu.VMEM((2, CH, D), jnp.float32),
        pltpu.VMEM((NBUF, CH, D), jnp.float32),
        pltpu.SemaphoreType.DMA((NBUF,)),
        pltpu.SemaphoreType.DMA((NBUF,)),
        pltpu.SemaphoreType.DMA((2,)),
        pltpu.SemaphoreType.DMA,
    ],
)
def _embed_kernel(tok_hbm, tokens_hbm, pos_hbm, out_hbm,
                  idx_refs, pos_v, rows_v, g_sem, o_sem, p_sem, i_sem):
    _body(tok_hbm, tokens_hbm, pos_hbm, out_hbm,
          idx_refs, pos_v, rows_v, g_sem, o_sem, p_sem, i_sem)


def kernel(tokens, tok_table, pos_table):
    return _embed_kernel(tok_table, tokens.astype(jnp.int32), pos_table)
